# baseline (speedup 1.0000x reference)
import functools

import jax
import jax.numpy as jnp
from jax import lax
from jax.experimental import pallas as pl
from jax.experimental.pallas import tpu as pltpu
from jax.experimental.pallas import tpu_sc as plsc

NC = 2
NS = 16
NW = NC * NS
L = 16


def _prep_body(uq_ref, wq_ref, bq_ref, wk_ref, pq_ref, qk2_ref):
    pq = jnp.dot(uq_ref[...], wq_ref[...].T,
                 preferred_element_type=jnp.float32) + bq_ref[...]
    pq_ref[...] = pq
    qk2_ref[...] = jnp.dot(pq, wk_ref[...], preferred_element_type=jnp.float32)


def _prep(uq, Wq, bq2, Wk, blk):
    U, D = uq.shape
    grid = (U // blk,)
    return pl.pallas_call(
        _prep_body,
        grid=grid,
        in_specs=[
            pl.BlockSpec((blk, D), lambda i: (i, 0)),
            pl.BlockSpec((D, D), lambda i: (0, 0)),
            pl.BlockSpec((1, D), lambda i: (0, 0)),
            pl.BlockSpec((D, D), lambda i: (0, 0)),
        ],
        out_specs=[
            pl.BlockSpec((blk, D), lambda i: (i, 0)),
            pl.BlockSpec((blk, D), lambda i: (i, 0)),
        ],
        out_shape=[
            jax.ShapeDtypeStruct((U, D), jnp.float32),
            jax.ShapeDtypeStruct((U, D), jnp.float32),
        ],
    )(uq, Wq, bq2, Wk)


def _combine_body(b_ref, asn_ref, pq_ref, wv_ref, bv_ref, bk_ref, out_ref):
    B = b_ref[0] + b_ref[1]
    asn = asn_ref[0] + asn_ref[1]
    A = asn[:, 0]
    S = asn[:, 1]
    n = asn[:, 2]
    pq = pq_ref[...]
    c = (pq * bk_ref[...]).sum(axis=1)
    t = c - S - n * c
    sumexp_chk = jnp.exp(jnp.log(A) + t)
    den = A + 1e-12 * jnp.exp(-t)
    X = jnp.dot(B, wv_ref[...].T, preferred_element_type=jnp.float32) \
        + A[:, None] * bv_ref[...]
    prof = X / den[:, None]
    prof = jnp.where(jnp.isinf(sumexp_chk)[:, None], 0.0, prof)
    out_ref[...] = jnp.where((n == 0.0)[:, None], pq, prof)


def _combine(b_p, asn_p, pq, Wv, bv2, bk2, blk):
    U, D = pq.shape
    grid = (U // blk,)
    return pl.pallas_call(
        _combine_body,
        grid=grid,
        in_specs=[
            pl.BlockSpec((NC, blk, D), lambda i: (0, i, 0)),
            pl.BlockSpec((NC, blk, L), lambda i: (0, i, 0)),
            pl.BlockSpec((blk, D), lambda i: (i, 0)),
            pl.BlockSpec((D, D), lambda i: (0, 0)),
            pl.BlockSpec((1, D), lambda i: (0, 0)),
            pl.BlockSpec((1, D), lambda i: (0, 0)),
        ],
        out_specs=pl.BlockSpec((blk, D), lambda i: (i, 0)),
        out_shape=jax.ShapeDtypeStruct((U, D), jnp.float32),
    )(b_p, asn_p, pq, Wv, bv2, bk2)


def _sc_body(U, D, E, CH, item_hbm, qk2_hbm, ui_hbm, ii_hbm,
             bout_hbm, asnout_hbm, *scr):
    ui4 = scr[0:4]
    ii4 = scr[4:8]
    g4 = scr[8:12]
    q4 = scr[12:16]
    asn4 = scr[16:20]
    b_sp = scr[20]
    asn_sp = scr[21]
    sem_i = scr[22:26]
    sem_gq = scr[26:30]
    sem_s = scr[30:34]

    cid = lax.axis_index("c")
    sid = lax.axis_index("s")
    wid = cid * NS + sid
    e_per_w = E // NW
    n_ch = e_per_w // CH
    u_per_t = U // NS

    zeros = jnp.zeros((L,), jnp.float32)

    def _zrow(i, _):
        for k in range(D // L):
            g4[0][i, pl.ds(k * L, L)] = zeros
        asn4[0][i, :] = zeros
        return ()

    lax.fori_loop(0, CH, _zrow, ())
    for j in range(u_per_t // CH):
        pltpu.sync_copy(g4[0], b_sp.at[pl.ds(sid * u_per_t + j * CH, CH)])
        pltpu.sync_copy(asn4[0], asn_sp.at[pl.ds(sid * u_per_t + j * CH, CH)])
    plsc.subcore_barrier()

    lane = lax.iota(jnp.int32, L)
    is0 = lane == 0
    is1 = lane == 1
    one2 = jnp.where(lane == 2, 1.0, 0.0)
    perms = [lane ^ (1 << k) for k in range(4)]

    gdn = lax.GatherDimensionNumbers(
        offset_dims=(), collapsed_slice_dims=(0,), start_index_map=(0,))

    def _shuffle(v, idx):
        return lax.gather(v, idx[:, None], gdn, slice_sizes=(1,),
                          mode=lax.GatherScatterMode.PROMISE_IN_BOUNDS)

    def _allsum(v):
        for p in perms:
            v = v + _shuffle(v, p)
        return v

    def idx_issue(ch, j):
        base = wid * e_per_w + ch * CH
        pltpu.async_copy(ui_hbm.at[pl.ds(base, CH)], ui4[j], sem_i[j])
        pltpu.async_copy(ii_hbm.at[pl.ds(base, CH)], ii4[j], sem_i[j])

    def idx_wait(j):
        pltpu.make_async_copy(ui_hbm.at[pl.ds(0, CH)], ui4[j], sem_i[j]).wait()
        pltpu.make_async_copy(ii_hbm.at[pl.ds(0, CH)], ii4[j], sem_i[j]).wait()

    def gather_issue(j):
        pltpu.async_copy(item_hbm.at[ii4[j]], g4[j], sem_gq[j])
        pltpu.async_copy(qk2_hbm.at[ui4[j]], q4[j], sem_gq[j])

    def gather_wait(j):
        pltpu.make_async_copy(item_hbm.at[ii4[j]], g4[j], sem_gq[j]).wait()
        pltpu.make_async_copy(qk2_hbm.at[ui4[j]], q4[j], sem_gq[j]).wait()

    def scatter_issue(j):
        pltpu.async_copy(g4[j], b_sp.at[ui4[j]], sem_s[j], add=True)
        pltpu.async_copy(asn4[j], asn_sp.at[ui4[j]], sem_s[j], add=True)

    def scatter_wait(j):
        pltpu.make_async_copy(g4[j], b_sp.at[ui4[j]], sem_s[j]).wait()
        pltpu.make_async_copy(asn4[j], asn_sp.at[ui4[j]], sem_s[j]).wait()

    def compute(j):
        gv, qv, av = g4[j], q4[j], asn4[j]

        def _group(g8, _):
            for r0 in range(L):
                r = g8 * L + r0
                gs = [gv[r, pl.ds(k * L, L)] for k in range(D // L)]
                qs = [qv[r, pl.ds(k * L, L)] for k in range(D // L)]
                prod = gs[0] * qs[0]
                for k in range(1, D // L):
                    prod = prod + gs[k] * qs[k]
                dot_b = _allsum(prod)
                ed_b = jnp.exp(dot_b)
                for k in range(D // L):
                    gv[r, pl.ds(k * L, L)] = ed_b * gs[k]
                asn = jnp.where(is0, ed_b, jnp.where(is1, dot_b, one2))
                av[r, :] = asn
            return ()

        lax.fori_loop(0, CH // L, _group, ())

    def step(ch, j, first, idx_pf, gather_pf):
        gather_wait(j)
        if not first:
            scatter_wait((j - 2) % 4)
        if idx_pf:
            idx_issue(ch + 2, (j + 2) % 4)
        if gather_pf:
            idx_wait((j + 1) % 4)
            gather_issue((j + 1) % 4)
        compute(j)
        scatter_issue(j)

    idx_issue(0, 0)
    idx_issue(1, 1)
    idx_wait(0)
    gather_issue(0)
    step(0, 0, True, True, True)
    step(1, 1, True, True, True)

    def _quad(i, _):
        for k in range(4):
            ch = 4 * i + 2 + k
            step(ch, (2 + k) % 4, False, True, True)
        return ()

    lax.fori_loop(0, (n_ch - 4) // 4, _quad, ())
    step(n_ch - 2, (n_ch - 2) % 4, False, False, True)
    step(n_ch - 1, (n_ch - 1) % 4, False, False, False)
    scatter_wait((n_ch - 2) % 4)
    scatter_wait((n_ch - 1) % 4)

    plsc.subcore_barrier()
    pltpu.sync_copy(b_sp.at[pl.ds(sid * u_per_t, u_per_t)],
                    bout_hbm.at[cid, pl.ds(sid * u_per_t, u_per_t)])
    pltpu.sync_copy(asn_sp.at[pl.ds(sid * u_per_t, u_per_t)],
                    asnout_hbm.at[cid, pl.ds(sid * u_per_t, u_per_t)])


def _sc_main(item_features, qk2, ui, ii, CH=64):
    NI, D = item_features.shape
    U = qk2.shape[0]
    E = ui.shape[0]
    mesh = plsc.VectorSubcoreMesh(core_axis_name="c", subcore_axis_name="s")
    body = functools.partial(_sc_body, U, D, E, CH)
    f = pl.kernel(
        body,
        out_type=[
            jax.ShapeDtypeStruct((NC, U, D), jnp.float32),
            jax.ShapeDtypeStruct((NC, U, L), jnp.float32),
        ],
        mesh=mesh,
        compiler_params=pltpu.CompilerParams(use_tc_tiling_on_sc=False),
        scratch_types=(
            [pltpu.VMEM((CH,), jnp.int32) for _ in range(4)]
            + [pltpu.VMEM((CH,), jnp.int32) for _ in range(4)]
            + [pltpu.VMEM((CH, D), jnp.float32) for _ in range(4)]
            + [pltpu.VMEM((CH, D), jnp.float32) for _ in range(4)]
            + [pltpu.VMEM((CH, L), jnp.float32) for _ in range(4)]
            + [pltpu.VMEM_SHARED((U, D), jnp.float32),
               pltpu.VMEM_SHARED((U, L), jnp.float32)]
            + [pltpu.SemaphoreType.DMA for _ in range(12)]
        ),
    )
    return f(item_features, qk2, ui, ii)


def kernel(user_queries, item_features, Wq, bq, Wk, bk, Wv, bv,
           batch_user_indices, batch_item_indices):
    U, D = user_queries.shape
    bq2 = bq.reshape(1, D)
    bk2 = bk.reshape(1, D)
    bv2 = bv.reshape(1, D)
    pq, qk2 = _prep(user_queries, Wq, bq2, Wk, blk=1024)
    b_p, asn_p = _sc_main(item_features, qk2,
                          batch_user_indices, batch_item_indices)
    return _combine(b_p, asn_p, pq, Wv, bv2, bk2, blk=1024)


# speedup vs baseline: 1.0134x; 1.0134x over previous
import functools

import jax
import jax.numpy as jnp
from jax import lax
from jax.experimental import pallas as pl
from jax.experimental.pallas import tpu as pltpu
from jax.experimental.pallas import tpu_sc as plsc

NC = 2
NS = 16
NW = NC * NS
L = 16


def _prep_body(uq_ref, wq_ref, bq_ref, wk_ref, pq_ref, qk2_ref):
    pq = jnp.dot(uq_ref[...], wq_ref[...].T,
                 preferred_element_type=jnp.float32) + bq_ref[...]
    pq_ref[...] = pq
    qk2_ref[...] = jnp.dot(pq, wk_ref[...], preferred_element_type=jnp.float32)


def _prep(uq, Wq, bq2, Wk, blk):
    U, D = uq.shape
    grid = (U // blk,)
    return pl.pallas_call(
        _prep_body,
        grid=grid,
        in_specs=[
            pl.BlockSpec((blk, D), lambda i: (i, 0)),
            pl.BlockSpec((D, D), lambda i: (0, 0)),
            pl.BlockSpec((1, D), lambda i: (0, 0)),
            pl.BlockSpec((D, D), lambda i: (0, 0)),
        ],
        out_specs=[
            pl.BlockSpec((blk, D), lambda i: (i, 0)),
            pl.BlockSpec((blk, D), lambda i: (i, 0)),
        ],
        out_shape=[
            jax.ShapeDtypeStruct((U, D), jnp.float32),
            jax.ShapeDtypeStruct((U, D), jnp.float32),
        ],
    )(uq, Wq, bq2, Wk)


def _combine_body(b_ref, asn_ref, pq_ref, wv_ref, bv_ref, bk_ref, out_ref):
    B = b_ref[0] + b_ref[1]
    asn = asn_ref[0] + asn_ref[1]
    A = asn[:, 0]
    S = asn[:, 1]
    n = asn[:, 2]
    pq = pq_ref[...]
    c = (pq * bk_ref[...]).sum(axis=1)
    t = c - S - n * c
    sumexp_chk = jnp.exp(jnp.log(A) + t)
    den = A + 1e-12 * jnp.exp(-t)
    X = jnp.dot(B, wv_ref[...].T, preferred_element_type=jnp.float32) \
        + A[:, None] * bv_ref[...]
    prof = X / den[:, None]
    prof = jnp.where(jnp.isinf(sumexp_chk)[:, None], 0.0, prof)
    out_ref[...] = jnp.where((n == 0.0)[:, None], pq, prof)


def _combine(b_p, asn_p, pq, Wv, bv2, bk2, blk):
    U, D = pq.shape
    grid = (U // blk,)
    return pl.pallas_call(
        _combine_body,
        grid=grid,
        in_specs=[
            pl.BlockSpec((NC, blk, D), lambda i: (0, i, 0)),
            pl.BlockSpec((NC, blk, L), lambda i: (0, i, 0)),
            pl.BlockSpec((blk, D), lambda i: (i, 0)),
            pl.BlockSpec((D, D), lambda i: (0, 0)),
            pl.BlockSpec((1, D), lambda i: (0, 0)),
            pl.BlockSpec((1, D), lambda i: (0, 0)),
        ],
        out_specs=pl.BlockSpec((blk, D), lambda i: (i, 0)),
        out_shape=jax.ShapeDtypeStruct((U, D), jnp.float32),
    )(b_p, asn_p, pq, Wv, bv2, bk2)


def _sc_body(U, D, E, CH, item_hbm, qk2_hbm, ui_hbm, ii_hbm,
             bout_hbm, asnout_hbm,
             ui0, ii0, g0, q0, ui1, ii1, g1, q1, asn_v,
             b_sp, asn_sp,
             sem_ui0, sem_ii0, sem_g0, sem_q0,
             sem_ui1, sem_ii1, sem_g1, sem_q1):
    cid = lax.axis_index("c")
    sid = lax.axis_index("s")
    wid = cid * NS + sid
    e_per_w = E // NW
    n_ch = e_per_w // CH
    u_per_t = U // NS

    zeros = jnp.zeros((L,), jnp.float32)

    def _zrow(i, _):
        for k in range(D // L):
            g0[i, pl.ds(k * L, L)] = zeros
        asn_v[i, :] = zeros
        return ()

    lax.fori_loop(0, CH, _zrow, ())
    for j in range(u_per_t // CH):
        pltpu.sync_copy(g0, b_sp.at[pl.ds(sid * u_per_t + j * CH, CH)])
        pltpu.sync_copy(asn_v, asn_sp.at[pl.ds(sid * u_per_t + j * CH, CH)])
    plsc.subcore_barrier()

    lane = lax.iota(jnp.int32, L)
    is0 = lane == 0
    is1 = lane == 1
    one2 = jnp.where(lane == 2, 1.0, 0.0)
    perms = [lane ^ (1 << k) for k in range(4)]

    gdn = lax.GatherDimensionNumbers(
        offset_dims=(), collapsed_slice_dims=(0,), start_index_map=(0,))

    def _shuffle(v, idx):
        return lax.gather(v, idx[:, None], gdn, slice_sizes=(1,),
                          mode=lax.GatherScatterMode.PROMISE_IN_BOUNDS)

    def _allsum(v):
        for p in perms:
            v = v + _shuffle(v, p)
        return v

    slots = [
        (ui0, ii0, g0, q0, sem_ui0, sem_ii0, sem_g0, sem_q0),
        (ui1, ii1, g1, q1, sem_ui1, sem_ii1, sem_g1, sem_q1),
    ]

    def idx_issue(ch, s):
        uiv, iiv, _, _, sui, sii, _, _ = s
        base = wid * e_per_w + ch * CH
        pltpu.async_copy(ui_hbm.at[pl.ds(base, CH)], uiv, sui)
        pltpu.async_copy(ii_hbm.at[pl.ds(base, CH)], iiv, sii)

    def idx_wait(s):
        uiv, iiv, _, _, sui, sii, _, _ = s
        pltpu.make_async_copy(ui_hbm.at[pl.ds(0, CH)], uiv, sui).wait()
        pltpu.make_async_copy(ii_hbm.at[pl.ds(0, CH)], iiv, sii).wait()

    def gather_issue(s):
        uiv, iiv, gv, qv, _, _, sg, sq = s
        pltpu.async_copy(item_hbm.at[iiv], gv, sg)
        pltpu.async_copy(qk2_hbm.at[uiv], qv, sq)

    def gather_wait(s):
        uiv, iiv, gv, qv, _, _, sg, sq = s
        pltpu.make_async_copy(item_hbm.at[iiv], gv, sg).wait()
        pltpu.make_async_copy(qk2_hbm.at[uiv], qv, sq).wait()

    def compute(s):
        uiv, _, gv, qv, _, _, _, _ = s

        def _group(g8, _):
            for j in range(L):
                r = g8 * L + j
                gs = [gv[r, pl.ds(k * L, L)] for k in range(D // L)]
                qs = [qv[r, pl.ds(k * L, L)] for k in range(D // L)]
                prod = gs[0] * qs[0]
                for k in range(1, D // L):
                    prod = prod + gs[k] * qs[k]
                dot_b = _allsum(prod)
                ed_b = jnp.exp(dot_b)
                for k in range(D // L):
                    gv[r, pl.ds(k * L, L)] = ed_b * gs[k]
                asn = jnp.where(is0, ed_b, jnp.where(is1, dot_b, one2))
                asn_v[r, :] = asn
            return ()

        lax.fori_loop(0, CH // L, _group, ())
        pltpu.sync_copy(gv, b_sp.at[uiv], add=True)
        pltpu.sync_copy(asn_v, asn_sp.at[uiv], add=True)

    idx_issue(0, slots[0])
    idx_wait(slots[0])
    gather_issue(slots[0])
    idx_issue(1, slots[1])

    def _pair(i, _):
        for b in (0, 1):
            ch = 2 * i + b
            s = slots[b]
            s_n = slots[1 - b]
            gather_wait(s)
            idx_wait(s_n)
            gather_issue(s_n)
            compute(s)
            idx_issue(ch + 2, s)
        return ()

    lax.fori_loop(0, (n_ch - 2) // 2, _pair, ())
    gather_wait(slots[0])
    idx_wait(slots[1])
    gather_issue(slots[1])
    compute(slots[0])
    gather_wait(slots[1])
    compute(slots[1])

    plsc.subcore_barrier()
    pltpu.sync_copy(b_sp.at[pl.ds(sid * u_per_t, u_per_t)],
                    bout_hbm.at[cid, pl.ds(sid * u_per_t, u_per_t)])
    pltpu.sync_copy(asn_sp.at[pl.ds(sid * u_per_t, u_per_t)],
                    asnout_hbm.at[cid, pl.ds(sid * u_per_t, u_per_t)])


def _sc_main(item_features, qk2, ui, ii, CH=128):
    NI, D = item_features.shape
    U = qk2.shape[0]
    E = ui.shape[0]
    mesh = plsc.VectorSubcoreMesh(core_axis_name="c", subcore_axis_name="s")
    body = functools.partial(_sc_body, U, D, E, CH)
    f = pl.kernel(
        body,
        out_type=[
            jax.ShapeDtypeStruct((NC, U, D), jnp.float32),
            jax.ShapeDtypeStruct((NC, U, L), jnp.float32),
        ],
        mesh=mesh,
        compiler_params=pltpu.CompilerParams(use_tc_tiling_on_sc=False),
        scratch_types=[
            pltpu.VMEM((CH,), jnp.int32),
            pltpu.VMEM((CH,), jnp.int32),
            pltpu.VMEM((CH, D), jnp.float32),
            pltpu.VMEM((CH, D), jnp.float32),
            pltpu.VMEM((CH,), jnp.int32),
            pltpu.VMEM((CH,), jnp.int32),
            pltpu.VMEM((CH, D), jnp.float32),
            pltpu.VMEM((CH, D), jnp.float32),
            pltpu.VMEM((CH, L), jnp.float32),
            pltpu.VMEM_SHARED((U, D), jnp.float32),
            pltpu.VMEM_SHARED((U, L), jnp.float32),
            pltpu.SemaphoreType.DMA,
            pltpu.SemaphoreType.DMA,
            pltpu.SemaphoreType.DMA,
            pltpu.SemaphoreType.DMA,
            pltpu.SemaphoreType.DMA,
            pltpu.SemaphoreType.DMA,
            pltpu.SemaphoreType.DMA,
            pltpu.SemaphoreType.DMA,
        ],
    )
    return f(item_features, qk2, ui, ii)


def kernel(user_queries, item_features, Wq, bq, Wk, bk, Wv, bv,
           batch_user_indices, batch_item_indices):
    U, D = user_queries.shape
    bq2 = bq.reshape(1, D)
    bk2 = bk.reshape(1, D)
    bv2 = bv.reshape(1, D)
    pq, qk2 = _prep(user_queries, Wq, bq2, Wk, blk=1024)
    b_p, asn_p = _sc_main(item_features, qk2,
                          batch_user_indices, batch_item_indices)
    return _combine(b_p, asn_p, pq, Wv, bv2, bk2, blk=1024)


# speedup vs baseline: 1.0207x; 1.0072x over previous
import functools

import jax
import jax.numpy as jnp
from jax import lax
from jax.experimental import pallas as pl
from jax.experimental.pallas import tpu as pltpu
from jax.experimental.pallas import tpu_sc as plsc

NC = 2
NS = 16
NW = NC * NS
L = 16


def _prep_body(uq_ref, wq_ref, bq_ref, wk_ref, pq_ref, qk2_ref):
    pq = jnp.dot(uq_ref[...], wq_ref[...].T,
                 preferred_element_type=jnp.float32) + bq_ref[...]
    pq_ref[...] = pq
    qk2_ref[...] = jnp.dot(pq, wk_ref[...], preferred_element_type=jnp.float32)


def _prep(uq, Wq, bq2, Wk, blk):
    U, D = uq.shape
    grid = (U // blk,)
    return pl.pallas_call(
        _prep_body,
        grid=grid,
        in_specs=[
            pl.BlockSpec((blk, D), lambda i: (i, 0)),
            pl.BlockSpec((D, D), lambda i: (0, 0)),
            pl.BlockSpec((1, D), lambda i: (0, 0)),
            pl.BlockSpec((D, D), lambda i: (0, 0)),
        ],
        out_specs=[
            pl.BlockSpec((blk, D), lambda i: (i, 0)),
            pl.BlockSpec((blk, D), lambda i: (i, 0)),
        ],
        out_shape=[
            jax.ShapeDtypeStruct((U, D), jnp.float32),
            jax.ShapeDtypeStruct((U, D), jnp.float32),
        ],
    )(uq, Wq, bq2, Wk)


def _combine_body(b_ref, asn_ref, pq_ref, wv_ref, bv_ref, bk_ref, out_ref):
    B = b_ref[0] + b_ref[1]
    asn = asn_ref[0] + asn_ref[1]
    A = asn[:, 0]
    S = asn[:, 1]
    n = asn[:, 2]
    pq = pq_ref[...]
    c = (pq * bk_ref[...]).sum(axis=1)
    t = c - S - n * c
    sumexp_chk = jnp.exp(jnp.log(A) + t)
    den = A + 1e-12 * jnp.exp(-t)
    X = jnp.dot(B, wv_ref[...].T, preferred_element_type=jnp.float32) \
        + A[:, None] * bv_ref[...]
    prof = X / den[:, None]
    prof = jnp.where(jnp.isinf(sumexp_chk)[:, None], 0.0, prof)
    out_ref[...] = jnp.where((n == 0.0)[:, None], pq, prof)


def _combine(b_p, asn_p, pq, Wv, bv2, bk2, blk):
    U, D = pq.shape
    grid = (U // blk,)
    return pl.pallas_call(
        _combine_body,
        grid=grid,
        in_specs=[
            pl.BlockSpec((NC, blk, D), lambda i: (0, i, 0)),
            pl.BlockSpec((NC, blk, L), lambda i: (0, i, 0)),
            pl.BlockSpec((blk, D), lambda i: (i, 0)),
            pl.BlockSpec((D, D), lambda i: (0, 0)),
            pl.BlockSpec((1, D), lambda i: (0, 0)),
            pl.BlockSpec((1, D), lambda i: (0, 0)),
        ],
        out_specs=pl.BlockSpec((blk, D), lambda i: (i, 0)),
        out_shape=jax.ShapeDtypeStruct((U, D), jnp.float32),
    )(b_p, asn_p, pq, Wv, bv2, bk2)


def _sc_body(U, D, E, CH, item_hbm, qk2_hbm, ui_hbm, ii_hbm,
             bout_hbm, asnout_hbm,
             ui0, ii0, g0, q0, ui1, ii1, g1, q1, asn_v,
             b_sp, asn_sp,
             sem_ui0, sem_ii0, sem_g0, sem_q0,
             sem_ui1, sem_ii1, sem_g1, sem_q1):
    cid = lax.axis_index("c")
    sid = lax.axis_index("s")
    wid = cid * NS + sid
    e_per_w = E // NW
    n_ch = e_per_w // CH
    u_per_t = U // NS

    zeros = jnp.zeros((L,), jnp.float32)

    def _zrow(i, _):
        for k in range(D // L):
            g0[i, pl.ds(k * L, L)] = zeros
        asn_v[i, :] = zeros
        return ()

    lax.fori_loop(0, CH, _zrow, ())
    for j in range(u_per_t // CH):
        pltpu.sync_copy(g0, b_sp.at[pl.ds(sid * u_per_t + j * CH, CH)])
        pltpu.sync_copy(asn_v, asn_sp.at[pl.ds(sid * u_per_t + j * CH, CH)])
    plsc.subcore_barrier()

    lane = lax.iota(jnp.int32, L)
    is0 = lane == 0
    is1 = lane == 1
    one2 = jnp.where(lane == 2, 1.0, 0.0)
    perms = [lane ^ (1 << k) for k in range(4)]

    gdn = lax.GatherDimensionNumbers(
        offset_dims=(), collapsed_slice_dims=(0,), start_index_map=(0,))

    def _shuffle(v, idx):
        return lax.gather(v, idx[:, None], gdn, slice_sizes=(1,),
                          mode=lax.GatherScatterMode.PROMISE_IN_BOUNDS)

    def _allsum(v):
        for p in perms:
            v = v + _shuffle(v, p)
        return v

    slots = [
        (ui0, ii0, g0, q0, sem_ui0, sem_ii0, sem_g0, sem_q0),
        (ui1, ii1, g1, q1, sem_ui1, sem_ii1, sem_g1, sem_q1),
    ]

    def idx_issue(ch, s):
        uiv, iiv, _, _, sui, sii, _, _ = s
        base = wid * e_per_w + ch * CH
        pltpu.async_copy(ui_hbm.at[pl.ds(base, CH)], uiv, sui)
        pltpu.async_copy(ii_hbm.at[pl.ds(base, CH)], iiv, sii)

    def idx_wait(s):
        uiv, iiv, _, _, sui, sii, _, _ = s
        pltpu.make_async_copy(ui_hbm.at[pl.ds(0, CH)], uiv, sui).wait()
        pltpu.make_async_copy(ii_hbm.at[pl.ds(0, CH)], iiv, sii).wait()

    def gather_issue(s):
        uiv, iiv, gv, qv, _, _, sg, sq = s
        pltpu.async_copy(item_hbm.at[iiv], gv, sg)
        pltpu.async_copy(qk2_hbm.at[uiv], qv, sq)

    def gather_wait(s):
        uiv, iiv, gv, qv, _, _, sg, sq = s
        pltpu.make_async_copy(item_hbm.at[iiv], gv, sg).wait()
        pltpu.make_async_copy(qk2_hbm.at[uiv], qv, sq).wait()

    def compute(s):
        uiv, _, gv, qv, _, _, sg, sq = s

        def _group(g8, _):
            for j in range(L):
                r = g8 * L + j
                gs = [gv[r, pl.ds(k * L, L)] for k in range(D // L)]
                qs = [qv[r, pl.ds(k * L, L)] for k in range(D // L)]
                prod = gs[0] * qs[0]
                for k in range(1, D // L):
                    prod = prod + gs[k] * qs[k]
                dot_b = _allsum(prod)
                ed_b = jnp.exp(dot_b)
                for k in range(D // L):
                    gv[r, pl.ds(k * L, L)] = ed_b * gs[k]
                asn = jnp.where(is0, ed_b, jnp.where(is1, dot_b, one2))
                asn_v[r, :] = asn
            return ()

        lax.fori_loop(0, CH // L, _group, ())
        pltpu.async_copy(gv, b_sp.at[uiv], sg, add=True)
        pltpu.async_copy(asn_v, asn_sp.at[uiv], sq, add=True)
        pltpu.make_async_copy(gv, b_sp.at[uiv], sg).wait()
        pltpu.make_async_copy(asn_v, asn_sp.at[uiv], sq).wait()

    idx_issue(0, slots[0])
    idx_wait(slots[0])
    gather_issue(slots[0])
    idx_issue(1, slots[1])

    def _pair(i, _):
        for b in (0, 1):
            ch = 2 * i + b
            s = slots[b]
            s_n = slots[1 - b]
            gather_wait(s)
            idx_wait(s_n)
            gather_issue(s_n)
            compute(s)
            idx_issue(ch + 2, s)
        return ()

    lax.fori_loop(0, (n_ch - 2) // 2, _pair, ())
    gather_wait(slots[0])
    idx_wait(slots[1])
    gather_issue(slots[1])
    compute(slots[0])
    gather_wait(slots[1])
    compute(slots[1])

    plsc.subcore_barrier()
    pltpu.sync_copy(b_sp.at[pl.ds(sid * u_per_t, u_per_t)],
                    bout_hbm.at[cid, pl.ds(sid * u_per_t, u_per_t)])
    pltpu.sync_copy(asn_sp.at[pl.ds(sid * u_per_t, u_per_t)],
                    asnout_hbm.at[cid, pl.ds(sid * u_per_t, u_per_t)])


def _sc_main(item_features, qk2, ui, ii, CH=128):
    NI, D = item_features.shape
    U = qk2.shape[0]
    E = ui.shape[0]
    mesh = plsc.VectorSubcoreMesh(core_axis_name="c", subcore_axis_name="s")
    body = functools.partial(_sc_body, U, D, E, CH)
    f = pl.kernel(
        body,
        out_type=[
            jax.ShapeDtypeStruct((NC, U, D), jnp.float32),
            jax.ShapeDtypeStruct((NC, U, L), jnp.float32),
        ],
        mesh=mesh,
        compiler_params=pltpu.CompilerParams(use_tc_tiling_on_sc=False),
        scratch_types=[
            pltpu.VMEM((CH,), jnp.int32),
            pltpu.VMEM((CH,), jnp.int32),
            pltpu.VMEM((CH, D), jnp.float32),
            pltpu.VMEM((CH, D), jnp.float32),
            pltpu.VMEM((CH,), jnp.int32),
            pltpu.VMEM((CH,), jnp.int32),
            pltpu.VMEM((CH, D), jnp.float32),
            pltpu.VMEM((CH, D), jnp.float32),
            pltpu.VMEM((CH, L), jnp.float32),
            pltpu.VMEM_SHARED((U, D), jnp.float32),
            pltpu.VMEM_SHARED((U, L), jnp.float32),
            pltpu.SemaphoreType.DMA,
            pltpu.SemaphoreType.DMA,
            pltpu.SemaphoreType.DMA,
            pltpu.SemaphoreType.DMA,
            pltpu.SemaphoreType.DMA,
            pltpu.SemaphoreType.DMA,
            pltpu.SemaphoreType.DMA,
            pltpu.SemaphoreType.DMA,
        ],
    )
    return f(item_features, qk2, ui, ii)


def kernel(user_queries, item_features, Wq, bq, Wk, bk, Wv, bv,
           batch_user_indices, batch_item_indices):
    U, D = user_queries.shape
    bq2 = bq.reshape(1, D)
    bk2 = bk.reshape(1, D)
    bv2 = bv.reshape(1, D)
    pq, qk2 = _prep(user_queries, Wq, bq2, Wk, blk=1024)
    b_p, asn_p = _sc_main(item_features, qk2,
                          batch_user_indices, batch_item_indices)
    return _combine(b_p, asn_p, pq, Wv, bv2, bk2, blk=1024)
